# Initial kernel scaffold; baseline (speedup 1.0000x reference)
import jax
import jax.numpy as jnp
from jax.experimental import pallas as pl


def kernel(x_enc, gate_w1, gate_w2, sw, sb, tw, tb):
    raise NotImplementedError("write your pallas kernel here")


# speedup vs baseline: 1.2570x; 1.2570x over previous
import functools

import jax
import jax.numpy as jnp
from jax.experimental import pallas as pl
from jax.experimental.pallas import tpu as pltpu

B = 2048
L = 512
D = 512
E = 8
H = 256
KER = 25
PAD = (KER - 1) // 2
BM = 128


def _build_avg_matrix():
    l = jnp.arange(L)
    idx = jnp.clip(l[:, None] + jnp.arange(-PAD, PAD + 1)[None, :], 0, L - 1)
    A = jnp.zeros((L, L), jnp.float32).at[l[:, None], idx].add(1.0 / KER)
    return A


def _gating_kernel(x_ref, w1_ref, w2_ref, gates_ref, loss_ref, imp_ref, load_ref):
    j = pl.program_id(0)
    x = x_ref[...]
    h = jnp.maximum(jnp.dot(x, w1_ref[...], preferred_element_type=jnp.float32), 0.0)
    logits = jnp.dot(h, w2_ref[...], preferred_element_type=jnp.float32)
    m = jnp.max(logits, axis=1, keepdims=True)
    p = jnp.exp(logits - m)
    probs = p / jnp.sum(p, axis=1, keepdims=True)
    idx = jax.lax.broadcasted_iota(jnp.int32, probs.shape, 1)
    v1 = jnp.max(probs, axis=1, keepdims=True)
    a1 = jnp.min(jnp.where(probs == v1, idx, E), axis=1, keepdims=True)
    masked = jnp.where(idx == a1, -jnp.inf, probs)
    v2 = jnp.max(masked, axis=1, keepdims=True)
    a2 = jnp.min(jnp.where(masked == v2, idx, E), axis=1, keepdims=True)
    denom = v1 + v2 + 1e-6
    g = jnp.where(idx == a1, v1 / denom, 0.0) + jnp.where(idx == a2, v2 / denom, 0.0)
    gates_ref[...] = g
    blk_imp = jnp.sum(g, axis=0, keepdims=True)
    blk_load = jnp.sum((g > 0).astype(jnp.float32), axis=0, keepdims=True)

    @pl.when(j == 0)
    def _():
        imp_ref[...] = blk_imp
        load_ref[...] = blk_load

    @pl.when(j > 0)
    def _():
        imp_ref[...] += blk_imp
        load_ref[...] += blk_load

    @pl.when(j == pl.num_programs(0) - 1)
    def _():
        def cv2(v):
            mu = jnp.mean(v)
            var = jnp.sum((v - mu) ** 2) / (E - 1)
            return var / (mu * mu + 1e-10)

        loss_ref[...] = jnp.reshape((cv2(imp_ref[...]) + cv2(load_ref[...])) * 1e-2,
                                    (1, 1))


def _fold_kernel(a_ref, sw_ref, tw_ref, sb_ref, tb_ref, u_ref, b_ref):
    swe = sw_ref[0]
    diff = (tw_ref[0] - swe).astype(jnp.bfloat16)
    a16 = a_ref[...].astype(jnp.bfloat16)
    fold = jax.lax.dot_general(a16, diff, (((0,), (1,)), ((), ())),
                               preferred_element_type=jnp.float32)
    u_ref[0] = (swe.T + fold).astype(jnp.bfloat16)
    b_ref[...] = sb_ref[...] + tb_ref[...]


def _dense_kernel(x_ref, g_ref, u_ref, b_ref, y_ref):
    x = x_ref[...].astype(jnp.bfloat16)
    g = g_ref[...]
    acc = jnp.dot(g, b_ref[...], preferred_element_type=jnp.float32)
    for e in range(E):
        pe = jnp.dot(x, u_ref[e], preferred_element_type=jnp.float32)
        acc = acc + g[:, e:e + 1] * pe
    y_ref[...] = acc


def kernel(x_enc, gate_w1, gate_w2, sw, sb, tw, tb):
    mean = x_enc[:, :, 0]
    A = _build_avg_matrix()

    nblk = B // BM
    gates, loss = pl.pallas_call(
        _gating_kernel,
        grid=(nblk,),
        in_specs=[
            pl.BlockSpec((BM, L), lambda j: (j, 0)),
            pl.BlockSpec((L, H), lambda j: (0, 0)),
            pl.BlockSpec((H, E), lambda j: (0, 0)),
        ],
        out_specs=[
            pl.BlockSpec((BM, E), lambda j: (j, 0)),
            pl.BlockSpec((1, 1), lambda j: (0, 0)),
        ],
        out_shape=[
            jax.ShapeDtypeStruct((B, E), jnp.float32),
            jax.ShapeDtypeStruct((1, 1), jnp.float32),
        ],
        scratch_shapes=[
            pltpu.VMEM((1, E), jnp.float32),
            pltpu.VMEM((1, E), jnp.float32),
        ],
    )(mean, gate_w1, gate_w2)

    U, bias = pl.pallas_call(
        _fold_kernel,
        grid=(E,),
        in_specs=[
            pl.BlockSpec((L, L), lambda e: (0, 0)),
            pl.BlockSpec((1, D, L), lambda e: (e, 0, 0)),
            pl.BlockSpec((1, D, L), lambda e: (e, 0, 0)),
            pl.BlockSpec((E, D), lambda e: (0, 0)),
            pl.BlockSpec((E, D), lambda e: (0, 0)),
        ],
        out_specs=[
            pl.BlockSpec((1, L, D), lambda e: (e, 0, 0)),
            pl.BlockSpec((E, D), lambda e: (0, 0)),
        ],
        out_shape=[
            jax.ShapeDtypeStruct((E, L, D), jnp.bfloat16),
            jax.ShapeDtypeStruct((E, D), jnp.float32),
        ],
    )(A, sw, tw, sb, tb)

    y = pl.pallas_call(
        _dense_kernel,
        grid=(nblk,),
        in_specs=[
            pl.BlockSpec((BM, L), lambda j: (j, 0)),
            pl.BlockSpec((BM, E), lambda j: (j, 0)),
            pl.BlockSpec((E, L, D), lambda j: (0, 0, 0)),
            pl.BlockSpec((E, D), lambda j: (0, 0)),
        ],
        out_specs=pl.BlockSpec((BM, D), lambda j: (j, 0)),
        out_shape=jax.ShapeDtypeStruct((B, D), jnp.float32),
    )(mean, gates, U, bias)

    return y[:, :, None], loss[0, 0]


# speedup vs baseline: 2.7801x; 2.2116x over previous
import functools

import jax
import jax.numpy as jnp
from jax.experimental import pallas as pl
from jax.experimental.pallas import tpu as pltpu

B = 2048
L = 512
D = 512
E = 8
H = 256
KER = 25
PAD = (KER - 1) // 2
BM = 128


def _avg_matrix_in_kernel():
    li = jax.lax.broadcasted_iota(jnp.int32, (L, L), 0)
    ji = jax.lax.broadcasted_iota(jnp.int32, (L, L), 1)
    band = (jnp.abs(li - ji) <= PAD).astype(jnp.float32)
    n0 = jnp.clip(PAD + 1 - li, 0, KER).astype(jnp.float32)
    n1 = jnp.clip(li - (L - 2 - PAD), 0, KER).astype(jnp.float32)
    n = jnp.where(ji == 0, n0, jnp.where(ji == L - 1, n1, band))
    return n * (1.0 / KER)


def _gating_kernel(x_ref, w1_ref, w2_ref, gates_ref, loss_ref, imp_ref, load_ref):
    j = pl.program_id(0)
    x = x_ref[...]
    h = jnp.maximum(jnp.dot(x, w1_ref[...], preferred_element_type=jnp.float32), 0.0)
    logits = jnp.dot(h, w2_ref[...], preferred_element_type=jnp.float32)
    m = jnp.max(logits, axis=1, keepdims=True)
    p = jnp.exp(logits - m)
    probs = p / jnp.sum(p, axis=1, keepdims=True)
    idx = jax.lax.broadcasted_iota(jnp.int32, probs.shape, 1)
    v1 = jnp.max(probs, axis=1, keepdims=True)
    a1 = jnp.min(jnp.where(probs == v1, idx, E), axis=1, keepdims=True)
    masked = jnp.where(idx == a1, -jnp.inf, probs)
    v2 = jnp.max(masked, axis=1, keepdims=True)
    a2 = jnp.min(jnp.where(masked == v2, idx, E), axis=1, keepdims=True)
    denom = v1 + v2 + 1e-6
    g = jnp.where(idx == a1, v1 / denom, 0.0) + jnp.where(idx == a2, v2 / denom, 0.0)
    gates_ref[...] = g
    blk_imp = jnp.sum(g, axis=0, keepdims=True)
    blk_load = jnp.sum((g > 0).astype(jnp.float32), axis=0, keepdims=True)

    @pl.when(j == 0)
    def _():
        imp_ref[...] = blk_imp
        load_ref[...] = blk_load

    @pl.when(j > 0)
    def _():
        imp_ref[...] += blk_imp
        load_ref[...] += blk_load

    @pl.when(j == pl.num_programs(0) - 1)
    def _():
        def cv2(v):
            mu = jnp.mean(v)
            var = jnp.sum((v - mu) ** 2) / (E - 1)
            return var / (mu * mu + 1e-10)

        loss_ref[...] = jnp.reshape((cv2(imp_ref[...]) + cv2(load_ref[...])) * 1e-2,
                                    (1, 1))


def _fold_kernel(sw_ref, tw_ref, sb_ref, tb_ref, u_ref, b_ref):
    swe = sw_ref[0]
    diff = (tw_ref[0] - swe).astype(jnp.bfloat16)
    a16 = _avg_matrix_in_kernel().astype(jnp.bfloat16)
    fold = jax.lax.dot_general(a16, diff, (((0,), (1,)), ((), ())),
                               preferred_element_type=jnp.float32)
    u_ref[0] = (swe.T + fold).astype(jnp.bfloat16)
    b_ref[...] = sb_ref[...] + tb_ref[...]


def _dense_kernel(x_ref, g_ref, u_ref, b_ref, y_ref):
    x = x_ref[...].astype(jnp.bfloat16)
    g = g_ref[...]
    acc = jnp.dot(g, b_ref[...], preferred_element_type=jnp.float32)
    for e in range(E):
        pe = jnp.dot(x, u_ref[e], preferred_element_type=jnp.float32)
        acc = acc + g[:, e:e + 1] * pe
    y_ref[...] = acc


def kernel(x_enc, gate_w1, gate_w2, sw, sb, tw, tb):
    mean = x_enc[:, :, 0]

    nblk = B // BM
    gates, loss = pl.pallas_call(
        _gating_kernel,
        grid=(nblk,),
        in_specs=[
            pl.BlockSpec((BM, L), lambda j: (j, 0)),
            pl.BlockSpec((L, H), lambda j: (0, 0)),
            pl.BlockSpec((H, E), lambda j: (0, 0)),
        ],
        out_specs=[
            pl.BlockSpec((BM, E), lambda j: (j, 0)),
            pl.BlockSpec((1, 1), lambda j: (0, 0)),
        ],
        out_shape=[
            jax.ShapeDtypeStruct((B, E), jnp.float32),
            jax.ShapeDtypeStruct((1, 1), jnp.float32),
        ],
        scratch_shapes=[
            pltpu.VMEM((1, E), jnp.float32),
            pltpu.VMEM((1, E), jnp.float32),
        ],
    )(mean, gate_w1, gate_w2)

    U, bias = pl.pallas_call(
        _fold_kernel,
        grid=(E,),
        in_specs=[
            pl.BlockSpec((1, D, L), lambda e: (e, 0, 0)),
            pl.BlockSpec((1, D, L), lambda e: (e, 0, 0)),
            pl.BlockSpec((E, D), lambda e: (0, 0)),
            pl.BlockSpec((E, D), lambda e: (0, 0)),
        ],
        out_specs=[
            pl.BlockSpec((1, L, D), lambda e: (e, 0, 0)),
            pl.BlockSpec((E, D), lambda e: (0, 0)),
        ],
        out_shape=[
            jax.ShapeDtypeStruct((E, L, D), jnp.bfloat16),
            jax.ShapeDtypeStruct((E, D), jnp.float32),
        ],
    )(sw, tw, sb, tb)

    y = pl.pallas_call(
        _dense_kernel,
        grid=(nblk,),
        in_specs=[
            pl.BlockSpec((BM, L), lambda j: (j, 0)),
            pl.BlockSpec((BM, E), lambda j: (j, 0)),
            pl.BlockSpec((E, L, D), lambda j: (0, 0, 0)),
            pl.BlockSpec((E, D), lambda j: (0, 0)),
        ],
        out_specs=pl.BlockSpec((BM, D), lambda j: (j, 0)),
        out_shape=jax.ShapeDtypeStruct((B, D), jnp.float32),
    )(mean, gates, U, bias)

    return y[:, :, None], loss[0, 0]


# speedup vs baseline: 3.7721x; 1.3568x over previous
import functools

import jax
import jax.numpy as jnp
from jax.experimental import pallas as pl
from jax.experimental.pallas import tpu as pltpu

B = 2048
L = 512
D = 512
E = 8
H = 256
KER = 25
PAD = (KER - 1) // 2
BM = 256


def _avg_matrix_in_kernel():
    li = jax.lax.broadcasted_iota(jnp.int32, (L, L), 0)
    ji = jax.lax.broadcasted_iota(jnp.int32, (L, L), 1)
    band = (jnp.abs(li - ji) <= PAD).astype(jnp.float32)
    n0 = jnp.clip(PAD + 1 - li, 0, KER).astype(jnp.float32)
    n1 = jnp.clip(li - (L - 2 - PAD), 0, KER).astype(jnp.float32)
    n = jnp.where(ji == 0, n0, jnp.where(ji == L - 1, n1, band))
    return n * (1.0 / KER)


def _gates_transposed(x, w1, w2):
    h = jnp.maximum(jnp.dot(x, w1, preferred_element_type=jnp.float32), 0.0)
    logits = jnp.dot(h, w2, preferred_element_type=jnp.float32)
    lt = jnp.transpose(logits)
    m = jnp.max(lt, axis=0, keepdims=True)
    p = jnp.exp(lt - m)
    probs = p / jnp.sum(p, axis=0, keepdims=True)
    idx = jax.lax.broadcasted_iota(jnp.int32, probs.shape, 0)
    v1 = jnp.max(probs, axis=0, keepdims=True)
    a1 = jnp.min(jnp.where(probs == v1, idx, E), axis=0, keepdims=True)
    masked = jnp.where(idx == a1, -jnp.inf, probs)
    v2 = jnp.max(masked, axis=0, keepdims=True)
    a2 = jnp.min(jnp.where(masked == v2, idx, E), axis=0, keepdims=True)
    denom = v1 + v2 + 1e-6
    return (jnp.where(idx == a1, v1 / denom, 0.0)
            + jnp.where(idx == a2, v2 / denom, 0.0))


def _loss_accumulate(gates_t, j, loss_ref, imp_ref, load_ref):
    blk_imp = jnp.sum(gates_t, axis=1, keepdims=True)
    blk_load = jnp.sum((gates_t > 0).astype(jnp.float32), axis=1, keepdims=True)

    @pl.when(j == 0)
    def _():
        imp_ref[...] = blk_imp
        load_ref[...] = blk_load

    @pl.when(j > 0)
    def _():
        imp_ref[...] += blk_imp
        load_ref[...] += blk_load

    @pl.when(j == pl.num_programs(0) - 1)
    def _():
        def cv2(v):
            mu = jnp.mean(v)
            var = jnp.sum((v - mu) ** 2) / (E - 1)
            return var / (mu * mu + 1e-10)

        loss_ref[...] = jnp.reshape((cv2(imp_ref[...]) + cv2(load_ref[...])) * 1e-2,
                                    (1, 1))


def _fold_kernel(sw_ref, tw_ref, sb_ref, tb_ref, u_ref, b_ref):
    swe = sw_ref[0]
    diff = (tw_ref[0] - swe).astype(jnp.bfloat16)
    a16 = _avg_matrix_in_kernel().astype(jnp.bfloat16)
    fold = jax.lax.dot_general(a16, diff, (((0,), (1,)), ((), ())),
                               preferred_element_type=jnp.float32)
    u_ref[0] = (swe.T + fold).astype(jnp.bfloat16)
    b_ref[...] = sb_ref[...] + tb_ref[...]


def _moe_kernel(x_ref, w1_ref, w2_ref, u_ref, b_ref, y_ref, loss_ref,
                imp_ref, load_ref):
    j = pl.program_id(0)
    x = x_ref[...]
    gates_t = _gates_transposed(x, w1_ref[...], w2_ref[...])
    _loss_accumulate(gates_t, j, loss_ref, imp_ref, load_ref)
    g = jnp.transpose(gates_t)
    x16 = x.astype(jnp.bfloat16)
    acc = jnp.dot(g, b_ref[...], preferred_element_type=jnp.float32)
    for e in range(E):
        pe = jnp.dot(x16, u_ref[e], preferred_element_type=jnp.float32)
        acc = acc + g[:, e:e + 1] * pe
    y_ref[...] = acc


def kernel(x_enc, gate_w1, gate_w2, sw, sb, tw, tb):
    mean = x_enc[:, :, 0]

    U, bias = pl.pallas_call(
        _fold_kernel,
        grid=(E,),
        in_specs=[
            pl.BlockSpec((1, D, L), lambda e: (e, 0, 0)),
            pl.BlockSpec((1, D, L), lambda e: (e, 0, 0)),
            pl.BlockSpec((E, D), lambda e: (0, 0)),
            pl.BlockSpec((E, D), lambda e: (0, 0)),
        ],
        out_specs=[
            pl.BlockSpec((1, L, D), lambda e: (e, 0, 0)),
            pl.BlockSpec((E, D), lambda e: (0, 0)),
        ],
        out_shape=[
            jax.ShapeDtypeStruct((E, L, D), jnp.bfloat16),
            jax.ShapeDtypeStruct((E, D), jnp.float32),
        ],
    )(sw, tw, sb, tb)

    nblk = B // BM
    y, loss = pl.pallas_call(
        _moe_kernel,
        grid=(nblk,),
        in_specs=[
            pl.BlockSpec((BM, L), lambda j: (j, 0)),
            pl.BlockSpec((L, H), lambda j: (0, 0)),
            pl.BlockSpec((H, E), lambda j: (0, 0)),
            pl.BlockSpec((E, L, D), lambda j: (0, 0, 0)),
            pl.BlockSpec((E, D), lambda j: (0, 0)),
        ],
        out_specs=[
            pl.BlockSpec((BM, D), lambda j: (j, 0)),
            pl.BlockSpec((1, 1), lambda j: (0, 0)),
        ],
        out_shape=[
            jax.ShapeDtypeStruct((B, D), jnp.float32),
            jax.ShapeDtypeStruct((1, 1), jnp.float32),
        ],
        scratch_shapes=[
            pltpu.VMEM((E, 1), jnp.float32),
            pltpu.VMEM((E, 1), jnp.float32),
        ],
    )(mean, gate_w1, gate_w2, U, bias)

    return y[:, :, None], loss[0, 0]


# speedup vs baseline: 3.8445x; 1.0192x over previous
import jax
import jax.numpy as jnp
from jax.experimental import pallas as pl
from jax.experimental.pallas import tpu as pltpu

B = 2048
L = 512
D = 512
E = 8
H = 256
KER = 25
PAD = (KER - 1) // 2
BM = 256


def _avg_matrix_in_kernel():
    li = jax.lax.broadcasted_iota(jnp.int32, (L, L), 0)
    ji = jax.lax.broadcasted_iota(jnp.int32, (L, L), 1)
    band = (jnp.abs(li - ji) <= PAD).astype(jnp.float32)
    n0 = jnp.clip(PAD + 1 - li, 0, KER).astype(jnp.float32)
    n1 = jnp.clip(li - (L - 2 - PAD), 0, KER).astype(jnp.float32)
    n = jnp.where(ji == 0, n0, jnp.where(ji == L - 1, n1, band))
    return n * (1.0 / KER)


def _gates_transposed(x, w1, w2):
    h = jnp.maximum(jnp.dot(x, w1, preferred_element_type=jnp.float32), 0.0)
    logits = jnp.dot(h, w2, preferred_element_type=jnp.float32)
    lt = jnp.transpose(logits)
    m = jnp.max(lt, axis=0, keepdims=True)
    p = jnp.exp(lt - m)
    probs = p / jnp.sum(p, axis=0, keepdims=True)
    idx = jax.lax.broadcasted_iota(jnp.int32, probs.shape, 0)
    v1 = jnp.max(probs, axis=0, keepdims=True)
    a1 = jnp.min(jnp.where(probs == v1, idx, E), axis=0, keepdims=True)
    masked = jnp.where(idx == a1, -jnp.inf, probs)
    v2 = jnp.max(masked, axis=0, keepdims=True)
    a2 = jnp.min(jnp.where(masked == v2, idx, E), axis=0, keepdims=True)
    denom = v1 + v2 + 1e-6
    return (jnp.where(idx == a1, v1 / denom, 0.0)
            + jnp.where(idx == a2, v2 / denom, 0.0))


def _loss_accumulate(gates_t, j, last, loss_ref, imp_ref, load_ref):
    blk_imp = jnp.sum(gates_t, axis=1, keepdims=True)
    blk_load = jnp.sum((gates_t > 0).astype(jnp.float32), axis=1, keepdims=True)

    @pl.when(j == 0)
    def _():
        imp_ref[...] = blk_imp
        load_ref[...] = blk_load

    @pl.when(j > 0)
    def _():
        imp_ref[...] += blk_imp
        load_ref[...] += blk_load

    @pl.when(last)
    def _():
        def cv2(v):
            mu = jnp.mean(v)
            var = jnp.sum((v - mu) ** 2) / (E - 1)
            return var / (mu * mu + 1e-10)

        loss_ref[...] = jnp.reshape((cv2(imp_ref[...]) + cv2(load_ref[...])) * 1e-2,
                                    (1, 1))


def _moe_kernel(x_ref, w1_ref, w2_ref, sw_ref, tw_ref, sb_ref, tb_ref,
                y_ref, loss_ref, u_ref, imp_ref, load_ref):
    p = pl.program_id(0)
    j = pl.program_id(1)

    @pl.when(p == 0)
    def _fold():
        swe = sw_ref[0]
        diff = (tw_ref[0] - swe).astype(jnp.bfloat16)
        a16 = _avg_matrix_in_kernel().astype(jnp.bfloat16)
        fold = jax.lax.dot_general(a16, diff, (((0,), (1,)), ((), ())),
                                   preferred_element_type=jnp.float32)
        u_ref[pl.ds(j * L, L), :] = (swe.T + fold).astype(jnp.bfloat16)

    @pl.when(p == 1)
    def _tokens():
        x = x_ref[...]
        gates_t = _gates_transposed(x, w1_ref[...], w2_ref[...])
        _loss_accumulate(gates_t, j,
                         jnp.logical_and(p == 1, j == pl.num_programs(1) - 1),
                         loss_ref, imp_ref, load_ref)
        g = jnp.transpose(gates_t)
        x16 = x.astype(jnp.bfloat16)
        g16 = g.astype(jnp.bfloat16)
        xg = jnp.concatenate([g16[:, e:e + 1] * x16 for e in range(E)], axis=1)
        bsum = sb_ref[...] + tb_ref[...]
        acc = jnp.dot(g, bsum, preferred_element_type=jnp.float32)
        acc = acc + jnp.dot(xg, u_ref[...], preferred_element_type=jnp.float32)
        y_ref[...] = acc


def kernel(x_enc, gate_w1, gate_w2, sw, sb, tw, tb):
    mean = x_enc[:, :, 0]
    nblk = B // BM

    y, loss = pl.pallas_call(
        _moe_kernel,
        grid=(2, nblk),
        in_specs=[
            pl.BlockSpec((BM, L), lambda p, j: (jnp.where(p == 1, j, 0), 0)),
            pl.BlockSpec((L, H), lambda p, j: (0, 0)),
            pl.BlockSpec((H, E), lambda p, j: (0, 0)),
            pl.BlockSpec((1, D, L), lambda p, j: (jnp.where(p == 0, j, 0), 0, 0)),
            pl.BlockSpec((1, D, L), lambda p, j: (jnp.where(p == 0, j, 0), 0, 0)),
            pl.BlockSpec((E, D), lambda p, j: (0, 0)),
            pl.BlockSpec((E, D), lambda p, j: (0, 0)),
        ],
        out_specs=[
            pl.BlockSpec((BM, D), lambda p, j: (jnp.where(p == 1, j, 0), 0)),
            pl.BlockSpec((1, 1), lambda p, j: (0, 0)),
        ],
        out_shape=[
            jax.ShapeDtypeStruct((B, D), jnp.float32),
            jax.ShapeDtypeStruct((1, 1), jnp.float32),
        ],
        scratch_shapes=[
            pltpu.VMEM((E * L, D), jnp.bfloat16),
            pltpu.VMEM((E, 1), jnp.float32),
            pltpu.VMEM((E, 1), jnp.float32),
        ],
    )(mean, gate_w1, gate_w2, sw, tw, sb, tb)

    return y[:, :, None], loss[0, 0]


# speedup vs baseline: 4.1986x; 1.0921x over previous
import jax
import jax.numpy as jnp
from jax.experimental import pallas as pl
from jax.experimental.pallas import tpu as pltpu

B = 2048
L = 512
D = 512
E = 8
H = 256
KER = 25
PAD = (KER - 1) // 2
BM = 256


def _avg_matrix_in_kernel():
    li = jax.lax.broadcasted_iota(jnp.int32, (L, L), 0)
    ji = jax.lax.broadcasted_iota(jnp.int32, (L, L), 1)
    band = (jnp.abs(li - ji) <= PAD).astype(jnp.float32)
    n0 = jnp.clip(PAD + 1 - li, 0, KER).astype(jnp.float32)
    n1 = jnp.clip(li - (L - 2 - PAD), 0, KER).astype(jnp.float32)
    n = jnp.where(ji == 0, n0, jnp.where(ji == L - 1, n1, band))
    return n * (1.0 / KER)


def _gates_transposed(x, w1, w2):
    h = jnp.maximum(jnp.dot(x, w1, preferred_element_type=jnp.float32), 0.0)
    logits = jnp.dot(h, w2, preferred_element_type=jnp.float32)
    lt = jnp.transpose(logits)
    m = jnp.max(lt, axis=0, keepdims=True)
    p = jnp.exp(lt - m)
    probs = p / jnp.sum(p, axis=0, keepdims=True)
    idx = jax.lax.broadcasted_iota(jnp.int32, probs.shape, 0)
    v1 = jnp.max(probs, axis=0, keepdims=True)
    a1 = jnp.min(jnp.where(probs == v1, idx, E), axis=0, keepdims=True)
    masked = jnp.where(idx == a1, -jnp.inf, probs)
    v2 = jnp.max(masked, axis=0, keepdims=True)
    a2 = jnp.min(jnp.where(masked == v2, idx, E), axis=0, keepdims=True)
    denom = v1 + v2 + 1e-6
    return (jnp.where(idx == a1, v1 / denom, 0.0)
            + jnp.where(idx == a2, v2 / denom, 0.0))


def _loss_accumulate(gates_t, j, loss_ref, imp_ref, load_ref):
    blk_imp = jnp.sum(gates_t, axis=1, keepdims=True)
    blk_load = jnp.sum((gates_t > 0).astype(jnp.float32), axis=1, keepdims=True)

    @pl.when(j == 0)
    def _():
        imp_ref[...] = blk_imp
        load_ref[...] = blk_load

    @pl.when(j > 0)
    def _():
        imp_ref[...] += blk_imp
        load_ref[...] += blk_load

    @pl.when(j == pl.num_programs(1) - 1)
    def _():
        def cv2(v):
            mu = jnp.mean(v)
            var = jnp.sum((v - mu) ** 2) / (E - 1)
            return var / (mu * mu + 1e-10)

        loss_ref[...] = jnp.reshape((cv2(imp_ref[...]) + cv2(load_ref[...])) * 1e-2,
                                    (1, 1))


def _moe_kernel(x_ref, w1_ref, w2_ref, sw_ref, tw_ref, sb_ref, tb_ref,
                y_ref, loss_ref, u_ref, x16_ref, g_ref, imp_ref, load_ref):
    p = pl.program_id(0)
    j = pl.program_id(1)

    @pl.when(p == 0)
    def _fold_and_gate():
        swe = sw_ref[0]
        diff = (tw_ref[0] - swe).astype(jnp.bfloat16)
        a16 = _avg_matrix_in_kernel().astype(jnp.bfloat16)
        fold = jax.lax.dot_general(a16, diff, (((0,), (1,)), ((), ())),
                                   preferred_element_type=jnp.float32)
        u_ref[pl.ds(j * L, L), :] = (swe.T + fold).astype(jnp.bfloat16)

        x = x_ref[...]
        gates_t = _gates_transposed(x, w1_ref[...], w2_ref[...])
        _loss_accumulate(gates_t, j, loss_ref, imp_ref, load_ref)
        g_ref[pl.ds(j * BM, BM), :] = jnp.transpose(gates_t)
        x16_ref[pl.ds(j * BM, BM), :] = x.astype(jnp.bfloat16)

    @pl.when(p == 1)
    def _combine():
        xb = x16_ref[pl.ds(j * BM, BM), :]
        g = g_ref[pl.ds(j * BM, BM), :]
        g16 = g.astype(jnp.bfloat16)
        xg = jnp.concatenate([g16[:, e:e + 1] * xb for e in range(E)], axis=1)
        bsum = sb_ref[...] + tb_ref[...]
        acc = jnp.dot(g, bsum, preferred_element_type=jnp.float32)
        acc = acc + jnp.dot(xg, u_ref[...], preferred_element_type=jnp.float32)
        y_ref[...] = acc


def kernel(x_enc, gate_w1, gate_w2, sw, sb, tw, tb):
    mean = x_enc[:, :, 0]
    nblk = B // BM
    assert nblk == E

    y, loss = pl.pallas_call(
        _moe_kernel,
        grid=(2, nblk),
        in_specs=[
            pl.BlockSpec((BM, L), lambda p, j: (jnp.where(p == 0, j, 0), 0)),
            pl.BlockSpec((L, H), lambda p, j: (0, 0)),
            pl.BlockSpec((H, E), lambda p, j: (0, 0)),
            pl.BlockSpec((1, D, L), lambda p, j: (jnp.where(p == 0, j, 0), 0, 0)),
            pl.BlockSpec((1, D, L), lambda p, j: (jnp.where(p == 0, j, 0), 0, 0)),
            pl.BlockSpec((E, D), lambda p, j: (0, 0)),
            pl.BlockSpec((E, D), lambda p, j: (0, 0)),
        ],
        out_specs=[
            pl.BlockSpec((BM, D), lambda p, j: (jnp.where(p == 1, j, 0), 0)),
            pl.BlockSpec((1, 1), lambda p, j: (0, 0)),
        ],
        out_shape=[
            jax.ShapeDtypeStruct((B, D), jnp.float32),
            jax.ShapeDtypeStruct((1, 1), jnp.float32),
        ],
        scratch_shapes=[
            pltpu.VMEM((E * L, D), jnp.bfloat16),
            pltpu.VMEM((B, L), jnp.bfloat16),
            pltpu.VMEM((B, E), jnp.float32),
            pltpu.VMEM((E, 1), jnp.float32),
            pltpu.VMEM((E, 1), jnp.float32),
        ],
    )(mean, gate_w1, gate_w2, sw, tw, sb, tb)

    return y[:, :, None], loss[0, 0]
